# baseline (speedup 1.0000x reference)
import functools

import jax
import jax.numpy as jnp
from jax import lax
from jax.experimental import pallas as pl
from jax.experimental.pallas import tpu as pltpu
from jax.experimental.pallas import tpu_sc as plsc

VOCAB = 100000
D = 64
B, V, C = 1024, 50, 20
ROWS = B * V
NW = 32
ROWS_PER_W = ROWS // NW
CHUNK = 32
G = CHUNK * C
NCHUNK = ROWS_PER_W // CHUNK
IDX_W = 128
IDX_ROWS = G // IDX_W


def _body(x_hbm, table_hbm, out_hbm, idx_v,
          rows0, rows1, out0, out1, sem0, sem1):
    nc = 2
    wid = lax.axis_index("s") * nc + lax.axis_index("c")
    rows_b = (rows0, rows1)
    out_b = (out0, out1)
    sem_b = (sem0, sem1)
    pltpu.sync_copy(x_hbm.at[wid], idx_v)

    def fire(chunk, buf):
        for j in range(IDX_ROWS):
            pltpu.async_copy(
                table_hbm.at[idx_v.at[chunk * IDX_ROWS + j]],
                rows_b[buf].at[pl.ds(j * IDX_W, IDX_W)], sem_b[buf])

    def drain(buf):
        pltpu.make_async_copy(
            table_hbm.at[pl.ds(0, G)], rows_b[buf], sem_b[buf]).wait()

    def step(chunk, buf):
        rows_v = rows_b[buf]
        out_v = out_b[buf]

        @pl.when(chunk + 1 < NCHUNK)
        def _():
            fire(chunk + 1, 1 - buf)

        drain(buf)

        def acc_body(r2, _):
            for h in range(2):
                r = r2 * 2 + h
                for d in range(D // 16):
                    acc = rows_v[r * C, pl.ds(d * 16, 16)]
                    for c in range(1, C):
                        acc = acc + rows_v[r * C + c, pl.ds(d * 16, 16)]
                    out_v[r2, pl.ds(h * D + d * 16, 16)] = acc
            return 0

        lax.fori_loop(0, CHUNK // 2, acc_body, 0)
        pltpu.sync_copy(out_v, out_hbm.at[pl.ds((wid * ROWS_PER_W
                                                 + chunk * CHUNK) // 2,
                                                CHUNK // 2)])

    fire(0, 0)

    def outer(g0, _):
        for b in range(2):
            step(g0 + b, b)
        return 0

    lax.fori_loop(0, NCHUNK // 2, lambda i, c: outer(i * 2, c), 0)


IDX_PLANE = ROWS_PER_W * C // IDX_W
IDX_PLANE_PAD = 256


@jax.jit
def kernel(x, table):
    xf = x.astype(jnp.int32).reshape(NW, IDX_PLANE, IDX_W)
    xf = jnp.pad(xf, ((0, 0), (0, IDX_PLANE_PAD - IDX_PLANE), (0, 0)))
    mesh = plsc.VectorSubcoreMesh(core_axis_name="c", subcore_axis_name="s")
    out = pl.kernel(
        _body,
        out_type=jax.ShapeDtypeStruct((ROWS // 2, 2 * D), jnp.float32),
        mesh=mesh,
        compiler_params=pltpu.CompilerParams(use_tc_tiling_on_sc=False),
        scratch_types=[
            pltpu.VMEM((IDX_PLANE_PAD, IDX_W), jnp.int32),
            pltpu.VMEM((G, D), jnp.float32),
            pltpu.VMEM((G, D), jnp.float32),
            pltpu.VMEM((CHUNK // 2, 2 * D), jnp.float32),
            pltpu.VMEM((CHUNK // 2, 2 * D), jnp.float32),
            pltpu.SemaphoreType.DMA,
            pltpu.SemaphoreType.DMA,
        ],
    )(xf, table)
    return out.reshape(B, V, D)


# speedup vs baseline: 1.0114x; 1.0114x over previous
import functools

import jax
import jax.numpy as jnp
from jax import lax
from jax.experimental import pallas as pl
from jax.experimental.pallas import tpu as pltpu
from jax.experimental.pallas import tpu_sc as plsc

VOCAB = 100000
D = 64
B, V, C = 1024, 50, 20
ROWS = B * V
NW = 32
ROWS_PER_W = ROWS // NW
CHUNK = 32
G = CHUNK * C
NCHUNK = ROWS_PER_W // CHUNK
IDX_W = 128
NBATCH = G // IDX_W
IDX_PER_W = ROWS_PER_W * C


def _body(x_hbm, table_hbm, out_hbm, idx_v,
          rows0, rows1, out0, out1, sem0, sem1):
    nc = 2
    wid = lax.axis_index("s") * nc + lax.axis_index("c")
    rows_b = (rows0, rows1)
    out_b = (out0, out1)
    sem_b = (sem0, sem1)
    pltpu.sync_copy(x_hbm.at[pl.ds(wid * IDX_PER_W, IDX_PER_W)], idx_v)

    def fire(chunk, buf):
        for j in range(NBATCH):
            pltpu.async_copy(
                table_hbm.at[idx_v.at[pl.ds(chunk * G + j * IDX_W, IDX_W)]],
                rows_b[buf].at[pl.ds(j * IDX_W, IDX_W)], sem_b[buf])

    def drain(buf):
        pltpu.make_async_copy(
            table_hbm.at[pl.ds(0, G)], rows_b[buf], sem_b[buf]).wait()

    def step(chunk, buf):
        rows_v = rows_b[buf]
        out_v = out_b[buf]

        @pl.when(chunk + 1 < NCHUNK)
        def _():
            fire(chunk + 1, 1 - buf)

        drain(buf)

        def acc_body(r, _):
            for d in range(D // 16):
                acc = rows_v[r * C, pl.ds(d * 16, 16)]
                for c in range(1, C):
                    acc = acc + rows_v[r * C + c, pl.ds(d * 16, 16)]
                out_v[pl.ds(r * D + d * 16, 16)] = acc
            return 0

        lax.fori_loop(0, CHUNK, acc_body, 0)
        pltpu.sync_copy(out_v, out_hbm.at[pl.ds((wid * ROWS_PER_W
                                                 + chunk * CHUNK) * D,
                                                CHUNK * D)])

    fire(0, 0)

    def outer(g0, _):
        for b in range(2):
            step(g0 + b, b)
        return 0

    lax.fori_loop(0, NCHUNK // 2, lambda i, c: outer(i * 2, c), 0)


@jax.jit
def kernel(x, table):
    xf = x.astype(jnp.int32).reshape(ROWS * C)
    mesh = plsc.VectorSubcoreMesh(core_axis_name="c", subcore_axis_name="s")
    out = pl.kernel(
        _body,
        out_type=jax.ShapeDtypeStruct((ROWS * D,), jnp.float32),
        mesh=mesh,
        compiler_params=pltpu.CompilerParams(use_tc_tiling_on_sc=False),
        scratch_types=[
            pltpu.VMEM((IDX_PER_W,), jnp.int32),
            pltpu.VMEM((G, D), jnp.float32),
            pltpu.VMEM((G, D), jnp.float32),
            pltpu.VMEM((CHUNK * D,), jnp.float32),
            pltpu.VMEM((CHUNK * D,), jnp.float32),
            pltpu.SemaphoreType.DMA,
            pltpu.SemaphoreType.DMA,
        ],
    )(xf, table)
    return out.reshape(B, V, D)


# speedup vs baseline: 1.4755x; 1.4589x over previous
import functools

import jax
import jax.numpy as jnp
from jax import lax
from jax.experimental import pallas as pl
from jax.experimental.pallas import tpu as pltpu
from jax.experimental.pallas import tpu_sc as plsc

VOCAB = 100000
D = 64
B, V, C = 1024, 50, 20
NW = 32
BW = B // NW
G = C * BW
CV = C * V


def _body(x_hbm, table_hbm, out_hbm, idx_v,
          rows0, rows1, out0, out1, sem0, sem1):
    nc = 2
    wid = lax.axis_index("s") * nc + lax.axis_index("c")
    b0 = wid * BW
    rows_b = (rows0, rows1)
    out_b = (out0, out1)
    sem_b = (sem0, sem1)
    pltpu.sync_copy(x_hbm.at[:, pl.ds(b0, BW)], idx_v)

    def fire(v, buf):
        for c in range(C):
            pltpu.async_copy(
                table_hbm.at[idx_v.at[c * V + v]],
                rows_b[buf].at[pl.ds(c * BW, BW)], sem_b[buf])

    def drain(buf):
        pltpu.make_async_copy(
            table_hbm.at[pl.ds(0, G)], rows_b[buf], sem_b[buf]).wait()

    def step(v, buf):
        rows_v = rows_b[buf]
        out_v = out_b[buf]

        @pl.when(v + 1 < V)
        def _():
            fire(v + 1, 1 - buf)

        drain(buf)

        def acc_body(bl, _):
            for dw in range(D // 16):
                acc = rows_v[bl, pl.ds(dw * 16, 16)]
                for c in range(1, C):
                    acc = acc + rows_v[c * BW + bl, pl.ds(dw * 16, 16)]
                out_v[bl, pl.ds(dw * 16, 16)] = acc
            return 0

        lax.fori_loop(0, BW, acc_body, 0)
        pltpu.sync_copy(out_v, out_hbm.at[v, pl.ds(b0, BW), :])

    fire(0, 0)

    def outer(v0, _):
        for p in range(2):
            step(v0 + p, p)
        return 0

    lax.fori_loop(0, V // 2, lambda i, c: outer(i * 2, c), 0)


@jax.jit
def kernel(x, table):
    xq = x.astype(jnp.int32).transpose(2, 1, 0).reshape(CV, B)
    mesh = plsc.VectorSubcoreMesh(core_axis_name="c", subcore_axis_name="s")
    out = pl.kernel(
        _body,
        out_type=jax.ShapeDtypeStruct((V, B, D), jnp.float32),
        mesh=mesh,
        compiler_params=pltpu.CompilerParams(use_tc_tiling_on_sc=False),
        scratch_types=[
            pltpu.VMEM((CV, BW), jnp.int32),
            pltpu.VMEM((G, D), jnp.float32),
            pltpu.VMEM((G, D), jnp.float32),
            pltpu.VMEM((BW, D), jnp.float32),
            pltpu.VMEM((BW, D), jnp.float32),
            pltpu.SemaphoreType.DMA,
            pltpu.SemaphoreType.DMA,
        ],
    )(xq, table)
    return out.transpose(1, 0, 2)
